# Initial kernel scaffold; baseline (speedup 1.0000x reference)
import jax
import jax.numpy as jnp
from jax.experimental import pallas as pl


def kernel(inputs, indices):
    raise NotImplementedError("write your pallas kernel here")


# speedup vs baseline: 101.1159x; 101.1159x over previous
import jax
import jax.numpy as jnp
from jax import lax
from jax.experimental import pallas as pl
from jax.experimental.pallas import tpu as pltpu
from jax.experimental.pallas import tpu_sc as plsc

B, C, H, W = 2, 96, 256, 256
HO = WO = 512
L = 16
NC, NS = 2, 16
NW = NC * NS
R = 16
IN_W = R * W
OUT_W = 2 * R * WO
BLOCKS_PER_PLANE = H // R
NTASK = B * C * BLOCKS_PER_PLANE
TPW = NTASK // NW
NBUF = 2
TOTAL_IN = B * C * H * W
TOTAL_OUT = B * C * HO * WO


def _unpool_body(vals_hbm, idx_hbm, out_hbm,
                 v0, v1, i0, i1, o0, o1,
                 sv0, sv1, si0, si1, so0, so1):
    vb, ib, ob = (v0, v1), (i0, i1), (o0, o1)
    sv, si, so = (sv0, sv1), (si0, si1), (so0, so1)

    wid = lax.axis_index("s") * NC + lax.axis_index("c")
    g0 = wid * TPW

    def start_in(b, g):
        pltpu.async_copy(vals_hbm.at[pl.ds(g * IN_W, IN_W)], vb[b], sv[b])
        pltpu.async_copy(idx_hbm.at[pl.ds(g * IN_W, IN_W)], ib[b], si[b])

    def wait_in(b, g):
        pltpu.make_async_copy(vals_hbm.at[pl.ds(g * IN_W, IN_W)], vb[b], sv[b]).wait()
        pltpu.make_async_copy(idx_hbm.at[pl.ds(g * IN_W, IN_W)], ib[b], si[b]).wait()

    def wait_out(b):
        pltpu.make_async_copy(ob[b], out_hbm.at[pl.ds(0, OUT_W)], so[b]).wait()

    for b in range(NBUF):
        start_in(b, g0 + b)

    def outer(step, carry):
        for b in range(NBUF):
            t = step * NBUF + b
            g = g0 + t
            wait_in(b, g)

            @pl.when(t >= NBUF)
            def _():
                wait_out(b)

            base = (g & (BLOCKS_PER_PLANE - 1)) * OUT_W
            vbuf, ibuf, obuf = vb[b], ib[b], ob[b]

            def inner(k, c):
                off = pl.multiple_of(k * L, L)
                v = vbuf[pl.ds(off, L)]
                ix = ibuf[pl.ds(off, L)]
                local = ix - base
                s_bit = local & 1
                r_bit = (local >> 9) & 1
                o00 = local - s_bit - (r_bit << 9)
                q = (r_bit << 1) | s_bit
                zero = jnp.zeros((L,), jnp.float32)
                plsc.store_scatter(obuf, [o00], jnp.where(q == 0, v, zero))
                plsc.store_scatter(obuf, [o00 + 1], jnp.where(q == 1, v, zero))
                plsc.store_scatter(obuf, [o00 + WO], jnp.where(q == 2, v, zero))
                plsc.store_scatter(obuf, [o00 + (WO + 1)], jnp.where(q == 3, v, zero))
                return c

            lax.fori_loop(0, IN_W // L, inner, 0)

            pltpu.async_copy(obuf, out_hbm.at[pl.ds(g * OUT_W, OUT_W)], so[b])

            @pl.when(t + NBUF < TPW)
            def _():
                start_in(b, g + NBUF)
        return carry

    lax.fori_loop(0, TPW // NBUF, outer, 0)
    for b in range(NBUF):
        wait_out(b)


_unpool_call = pl.kernel(
    _unpool_body,
    out_type=jax.ShapeDtypeStruct((TOTAL_OUT,), jnp.float32),
    mesh=plsc.VectorSubcoreMesh(
        core_axis_name="c", subcore_axis_name="s",
        num_cores=NC, num_subcores=NS),
    compiler_params=pltpu.CompilerParams(needs_layout_passes=False),
    scratch_types=[
        pltpu.VMEM((IN_W,), jnp.float32),
        pltpu.VMEM((IN_W,), jnp.float32),
        pltpu.VMEM((IN_W,), jnp.int32),
        pltpu.VMEM((IN_W,), jnp.int32),
        pltpu.VMEM((OUT_W,), jnp.float32),
        pltpu.VMEM((OUT_W,), jnp.float32),
        pltpu.SemaphoreType.DMA,
        pltpu.SemaphoreType.DMA,
        pltpu.SemaphoreType.DMA,
        pltpu.SemaphoreType.DMA,
        pltpu.SemaphoreType.DMA,
        pltpu.SemaphoreType.DMA,
    ],
)


def kernel(inputs, indices):
    vals = inputs.reshape(TOTAL_IN)
    idx = indices.reshape(TOTAL_IN)
    out = _unpool_call(vals, idx)
    return out.reshape(B, C, HO, WO)


# speedup vs baseline: 181.7488x; 1.7974x over previous
import jax
import jax.numpy as jnp
from jax import lax
from jax.experimental import pallas as pl
from jax.experimental.pallas import tpu as pltpu
from jax.experimental.pallas import tpu_sc as plsc

B, C, H, W = 2, 96, 256, 256
HO = WO = 512
L = 16
NC, NS = 2, 16
NW = NC * NS
R = 16
BLOCKS_PER_PLANE = H // R
PLANES_PER_WORKER = (B * C) // NW
TPW = PLANES_PER_WORKER * BLOCKS_PER_PLANE
NBUF = 2


def _unpool_body(vals_hbm, idx_hbm, out_hbm,
                 v0, v1, i0, i1, o0, o1,
                 sv0, sv1, si0, si1, so0, so1):
    vb, ib, ob = (v0, v1), (i0, i1), (o0, o1)
    sv, si, so = (sv0, sv1), (si0, si1), (so0, so1)

    wid = lax.axis_index("s") * NC + lax.axis_index("c")
    b_idx = wid >> 4
    c_base = (wid & 15) * PLANES_PER_WORKER

    def in_copy(slot, t):
        p = t >> 4
        blk = t & 15
        src_v = vals_hbm.at[b_idx, c_base + p, pl.ds(blk * R, R), :]
        src_i = idx_hbm.at[b_idx, c_base + p, pl.ds(blk * R, R), :]
        return (pltpu.make_async_copy(src_v, vb[slot], sv[slot]),
                pltpu.make_async_copy(src_i, ib[slot], si[slot]))

    def start_in(slot, t):
        for c in in_copy(slot, t):
            c.start()

    def wait_in(slot, t):
        for c in in_copy(slot, t):
            c.wait()

    def out_copy(slot, t):
        p = t >> 4
        blk = t & 15
        dst = out_hbm.at[b_idx, c_base + p, pl.ds(blk * 2 * R, 2 * R), :]
        return pltpu.make_async_copy(ob[slot], dst, so[slot])

    for slot in range(NBUF):
        start_in(slot, slot)

    def outer(step, carry):
        for slot in range(NBUF):
            t = step * NBUF + slot
            wait_in(slot, t)

            @pl.when(t >= NBUF)
            def _():
                out_copy(slot, t - NBUF).wait()

            blk = t & 15
            row0 = blk * 2 * R
            vbuf, ibuf, obuf = vb[slot], ib[slot], ob[slot]
            zero = jnp.zeros((L,), jnp.float32)

            def inner(k, c):
                for j in range(W // L):
                    v = vbuf[k, pl.ds(j * L, L)]
                    ix = ibuf[k, pl.ds(j * L, L)]
                    lr = (ix >> 9) - row0
                    lc = ix & 511
                    r_bit = lr & 1
                    s_bit = lc & 1
                    lr0 = lr - r_bit
                    lc0 = lc - s_bit
                    q = (r_bit << 1) | s_bit
                    plsc.store_scatter(obuf, [lr0, lc0],
                                       jnp.where(q == 0, v, zero))
                    plsc.store_scatter(obuf, [lr0, lc0 + 1],
                                       jnp.where(q == 1, v, zero))
                    plsc.store_scatter(obuf, [lr0 + 1, lc0],
                                       jnp.where(q == 2, v, zero))
                    plsc.store_scatter(obuf, [lr0 + 1, lc0 + 1],
                                       jnp.where(q == 3, v, zero))
                return c

            lax.fori_loop(0, R, inner, 0)

            out_copy(slot, t).start()

            @pl.when(t + NBUF < TPW)
            def _():
                start_in(slot, t + NBUF)
        return carry

    lax.fori_loop(0, TPW // NBUF, outer, 0)
    for slot in range(NBUF):
        out_copy(slot, TPW - NBUF + slot).wait()


_unpool_call = pl.kernel(
    _unpool_body,
    out_type=jax.ShapeDtypeStruct((B, C, HO, WO), jnp.float32),
    mesh=plsc.VectorSubcoreMesh(
        core_axis_name="c", subcore_axis_name="s",
        num_cores=NC, num_subcores=NS),
    compiler_params=pltpu.CompilerParams(
        needs_layout_passes=False, use_tc_tiling_on_sc=True),
    scratch_types=[
        pltpu.VMEM((R, W), jnp.float32),
        pltpu.VMEM((R, W), jnp.float32),
        pltpu.VMEM((R, W), jnp.int32),
        pltpu.VMEM((R, W), jnp.int32),
        pltpu.VMEM((2 * R, WO), jnp.float32),
        pltpu.VMEM((2 * R, WO), jnp.float32),
        pltpu.SemaphoreType.DMA,
        pltpu.SemaphoreType.DMA,
        pltpu.SemaphoreType.DMA,
        pltpu.SemaphoreType.DMA,
        pltpu.SemaphoreType.DMA,
        pltpu.SemaphoreType.DMA,
    ],
)


def kernel(inputs, indices):
    return _unpool_call(inputs, indices)


# speedup vs baseline: 283.3204x; 1.5589x over previous
import jax
import jax.numpy as jnp
from jax import lax
from jax.experimental import pallas as pl
from jax.experimental.pallas import tpu as pltpu
from jax.experimental.pallas import tpu_sc as plsc

B, C, H, W = 2, 96, 256, 256
HO = WO = 512
L = 16
NC, NS = 2, 16
NW = NC * NS
R = 16
IN_W = R * W
OUT_W = 2 * R * WO
BLOCKS_PER_PLANE = H // R
PLANES_PER_WORKER = (B * C) // NW
TPW = PLANES_PER_WORKER * BLOCKS_PER_PLANE
NBUF = 2


def _unpool_body(vals_hbm, idx_hbm, out_hbm,
                 v0, v1, i0, i1, o0, o1,
                 sv0, sv1, si0, si1, so0, so1):
    vb, ib, ob = (v0, v1), (i0, i1), (o0, o1)
    sv, si, so = (sv0, sv1), (si0, si1), (so0, so1)

    wid = lax.axis_index("s") * NC + lax.axis_index("c")
    b_idx = wid >> 4
    c_base = (wid & 15) * PLANES_PER_WORKER

    def in_copy(slot, t):
        p = t >> 4
        blk = t & 15
        src_v = vals_hbm.at[b_idx, c_base + p, pl.ds(blk * R, R), :]
        src_i = idx_hbm.at[b_idx, c_base + p, pl.ds(blk * R, R), :]
        return (pltpu.make_async_copy(src_v, vb[slot], sv[slot]),
                pltpu.make_async_copy(src_i, ib[slot], si[slot]))

    def start_in(slot, t):
        for c in in_copy(slot, t):
            c.start()

    def wait_in(slot, t):
        for c in in_copy(slot, t):
            c.wait()

    def out_copy(slot, t):
        p = t >> 4
        blk = t & 15
        dst = out_hbm.at[b_idx, c_base + p, pl.ds(blk * 2 * R, 2 * R), :]
        return pltpu.make_async_copy(ob[slot], dst, so[slot])

    for slot in range(NBUF):
        start_in(slot, slot)

    def outer(step, carry):
        for slot in range(NBUF):
            t = step * NBUF + slot
            wait_in(slot, t)

            @pl.when(t >= NBUF)
            def _():
                out_copy(slot, t - NBUF).wait()

            blk = t & 15
            row0 = blk * 2 * R
            vbuf, ibuf, obuf = vb[slot], ib[slot], ob[slot]
            zero = jnp.zeros((L,), jnp.float32)

            @plsc.parallel_loop(0, IN_W // L, unroll=8)
            def inner(k):
                kr = k >> 4
                kc = pl.multiple_of((k & 15) * L, L)
                v = vbuf[kr, pl.ds(kc, L)]
                ix = ibuf[kr, pl.ds(kc, L)]
                lr = (ix >> 9) - row0
                lc = ix & 511
                r_bit = lr & 1
                s_bit = lc & 1
                lr0 = lr - r_bit
                lc0 = lc - s_bit
                q = (r_bit << 1) | s_bit
                plsc.store_scatter(obuf, [lr0, lc0],
                                   jnp.where(q == 0, v, zero))
                plsc.store_scatter(obuf, [lr0, lc0 + 1],
                                   jnp.where(q == 1, v, zero))
                plsc.store_scatter(obuf, [lr0 + 1, lc0],
                                   jnp.where(q == 2, v, zero))
                plsc.store_scatter(obuf, [lr0 + 1, lc0 + 1],
                                   jnp.where(q == 3, v, zero))

            out_copy(slot, t).start()

            @pl.when(t + NBUF < TPW)
            def _():
                start_in(slot, t + NBUF)
        return carry

    lax.fori_loop(0, TPW // NBUF, outer, 0)
    for slot in range(NBUF):
        out_copy(slot, TPW - NBUF + slot).wait()


_unpool_call = pl.kernel(
    _unpool_body,
    out_type=jax.ShapeDtypeStruct((B, C, HO, WO), jnp.float32),
    mesh=plsc.VectorSubcoreMesh(
        core_axis_name="c", subcore_axis_name="s",
        num_cores=NC, num_subcores=NS),
    compiler_params=pltpu.CompilerParams(
        needs_layout_passes=False, use_tc_tiling_on_sc=True),
    scratch_types=[
        pltpu.VMEM((R, W), jnp.float32),
        pltpu.VMEM((R, W), jnp.float32),
        pltpu.VMEM((R, W), jnp.int32),
        pltpu.VMEM((R, W), jnp.int32),
        pltpu.VMEM((2 * R, WO), jnp.float32),
        pltpu.VMEM((2 * R, WO), jnp.float32),
        pltpu.SemaphoreType.DMA,
        pltpu.SemaphoreType.DMA,
        pltpu.SemaphoreType.DMA,
        pltpu.SemaphoreType.DMA,
        pltpu.SemaphoreType.DMA,
        pltpu.SemaphoreType.DMA,
    ],
)


def kernel(inputs, indices):
    return _unpool_call(inputs, indices)


# speedup vs baseline: 453.0120x; 1.5989x over previous
import jax
import jax.numpy as jnp
from jax import lax
from jax.experimental import pallas as pl
from jax.experimental.pallas import tpu as pltpu
from jax.experimental.pallas import tpu_sc as plsc

B, C, H, W = 2, 96, 256, 256
HO = WO = 512
L = 16
NC, NS = 2, 16
NW = NC * NS
R = 16
IN_W = R * W
OUT_W = 2 * R * WO
BLOCKS_PER_PLANE = H // R
PLANES_PER_WORKER = (B * C) // NW
TPW = PLANES_PER_WORKER * BLOCKS_PER_PLANE
NBUF = 2


def _unpool_body(vals_hbm, idx_hbm, out_hbm,
                 v0, v1, i0, i1, o0, o1,
                 sv0, sv1, si0, si1, so0, so1):
    vb, ib, ob = (v0, v1), (i0, i1), (o0, o1)
    sv, si, so = (sv0, sv1), (si0, si1), (so0, so1)

    wid = lax.axis_index("s") * NC + lax.axis_index("c")
    b_idx = wid >> 4
    c_base = (wid & 15) * PLANES_PER_WORKER

    def in_copy(slot, t):
        p = t >> 4
        blk = t & 15
        src_v = vals_hbm.at[b_idx, c_base + p, pl.ds(blk * R, R), :]
        src_i = idx_hbm.at[b_idx, c_base + p, pl.ds(blk * R, R), :]
        return (pltpu.make_async_copy(src_v, vb[slot], sv[slot]),
                pltpu.make_async_copy(src_i, ib[slot], si[slot]))

    def start_in(slot, t):
        for c in in_copy(slot, t):
            c.start()

    def wait_in(slot, t):
        for c in in_copy(slot, t):
            c.wait()

    def out_copy(slot, t):
        p = t >> 4
        blk = t & 15
        dst = out_hbm.at[b_idx, c_base + p, pl.ds(blk * 2 * R, 2 * R), :]
        return pltpu.make_async_copy(ob[slot], dst, so[slot])

    for slot in range(NBUF):
        start_in(slot, slot)

    def outer(step, carry):
        for slot in range(NBUF):
            t = step * NBUF + slot
            wait_in(slot, t)

            @pl.when(t >= NBUF)
            def _():
                out_copy(slot, t - NBUF).wait()

            blk = t & 15
            row0 = blk * 2 * R
            vbuf, ibuf, obuf = vb[slot], ib[slot], ob[slot]
            zero = jnp.zeros((L,), jnp.float32)

            @plsc.parallel_loop(0, R, unroll=1)
            def inner(i_row):
                or0 = 2 * i_row
                or1 = or0 + 1
                for jj in range(0, WO, L):
                    obuf[or0, pl.ds(jj, L)] = zero
                    obuf[or1, pl.ds(jj, L)] = zero
                for jv in range(0, W, L):
                    v = vbuf[i_row, pl.ds(jv, L)]
                    ix = ibuf[i_row, pl.ds(jv, L)]
                    lr = (ix >> 9) - row0
                    lc = ix & 511
                    plsc.store_scatter(obuf, [lr, lc], v)

            out_copy(slot, t).start()

            @pl.when(t + NBUF < TPW)
            def _():
                start_in(slot, t + NBUF)
        return carry

    lax.fori_loop(0, TPW // NBUF, outer, 0)
    for slot in range(NBUF):
        out_copy(slot, TPW - NBUF + slot).wait()


_unpool_call = pl.kernel(
    _unpool_body,
    out_type=jax.ShapeDtypeStruct((B, C, HO, WO), jnp.float32),
    mesh=plsc.VectorSubcoreMesh(
        core_axis_name="c", subcore_axis_name="s",
        num_cores=NC, num_subcores=NS),
    compiler_params=pltpu.CompilerParams(
        needs_layout_passes=False, use_tc_tiling_on_sc=True),
    scratch_types=[
        pltpu.VMEM((R, W), jnp.float32),
        pltpu.VMEM((R, W), jnp.float32),
        pltpu.VMEM((R, W), jnp.int32),
        pltpu.VMEM((R, W), jnp.int32),
        pltpu.VMEM((2 * R, WO), jnp.float32),
        pltpu.VMEM((2 * R, WO), jnp.float32),
        pltpu.SemaphoreType.DMA,
        pltpu.SemaphoreType.DMA,
        pltpu.SemaphoreType.DMA,
        pltpu.SemaphoreType.DMA,
        pltpu.SemaphoreType.DMA,
        pltpu.SemaphoreType.DMA,
    ],
)


def kernel(inputs, indices):
    return _unpool_call(inputs, indices)
